# baseline (speedup 1.0000x reference)
import functools

import jax
import jax.numpy as jnp
from jax import lax
from jax.experimental import pallas as pl
from jax.experimental.pallas import tpu as pltpu
from jax.experimental.pallas import tpu_sc as plsc

N = 10000
E = 320000
D = 128

NC = 2
NS = 16
NW = NC * NS
L = 16

CH = 64
HNB = 2
SNB = 4
CPW = 160
EPAD = NW * CPW * CH
CW0 = 84
CW1 = 2 * CPW - CW0
NPAD = 10240
RPT = NPAD // NS


def _hist_body(dst_hbm, ones_hbm, zeros_hbm, out_hbm,
               dst_v0, dst_v1, ones_v0, ones_v1, acc_sh,
               si0, si1, ss0, ss1):
  dst_v = [dst_v0, dst_v1]
  ones_v = [ones_v0, ones_v1]
  sem_i = [si0, si1]
  sem_s = [ss0, ss1]
  c = lax.axis_index("c")
  s = lax.axis_index("s")
  wid = c * NS + s
  base0 = wid * CPW
  pltpu.sync_copy(zeros_hbm.at[pl.ds(s * RPT, RPT)],
                  acc_sh.at[pl.ds(s * RPT, RPT)])
  for b in range(HNB):
    pltpu.sync_copy(ones_hbm, ones_v[b])
  plsc.subcore_barrier()

  for b in range(HNB):
    eb = (base0 + b) * CH
    pltpu.async_copy(dst_hbm.at[pl.ds(eb, CH)], dst_v[b], sem_i[b])

  def group_body(i, _):
    j0 = i * HNB
    for b in range(HNB):
      pltpu.make_async_copy(
          dst_hbm.at[pl.ds(0, CH)], dst_v[b], sem_i[b]).wait()
      pltpu.async_copy(ones_v[b], acc_sh.at[dst_v[b]], sem_s[b], add=True)
    for b in range(HNB):
      pltpu.make_async_copy(
          ones_v[b], acc_sh.at[dst_v[b]], sem_s[b]).wait()
      nxt = j0 + HNB + b

      @pl.when(nxt < CPW)
      def _():
        eb = (base0 + nxt) * CH
        pltpu.async_copy(dst_hbm.at[pl.ds(eb, CH)], dst_v[b], sem_i[b])
    return 0

  lax.fori_loop(0, CPW // HNB, group_body, 0)
  plsc.subcore_barrier()
  pltpu.sync_copy(acc_sh.at[pl.ds(s * RPT, RPT)],
                  out_hbm.at[c, pl.ds(s * RPT, RPT)])


def _scatter_body(g_hbm, src_hbm, dst_hbm, zeros_hbm, out_hbm, *rest):
  src_v = list(rest[0:SNB])
  dst_v = list(rest[SNB:2 * SNB])
  rows_v = list(rest[2 * SNB:3 * SNB])
  acc_sh = rest[3 * SNB]
  sem_i = list(rest[3 * SNB + 1:3 * SNB + 1 + SNB])
  sem_g = list(rest[3 * SNB + 1 + SNB:3 * SNB + 1 + 2 * SNB])
  sem_s = list(rest[3 * SNB + 1 + 2 * SNB:3 * SNB + 1 + 3 * SNB])
  c = lax.axis_index("c")
  s = lax.axis_index("s")
  base0 = jnp.where(c == 0, s * CW0, NS * CW0 + s * CW1)
  cnt = jnp.where(c == 0, CW0, CW1)

  pltpu.sync_copy(zeros_hbm.at[pl.ds(s * RPT, RPT)],
                  acc_sh.at[pl.ds(s * RPT, RPT)])
  plsc.subcore_barrier()

  for b in range(SNB):
    eb = (base0 + b) * CH
    pltpu.async_copy(src_hbm.at[pl.ds(eb, CH)], src_v[b], sem_i[b])
    pltpu.async_copy(dst_hbm.at[pl.ds(eb, CH)], dst_v[b], sem_i[b])

  def group_body(i, _):
    j0 = i * SNB
    for b in range(SNB):
      pltpu.make_async_copy(
          src_hbm.at[pl.ds(0, CH)], src_v[b], sem_i[b]).wait()
      pltpu.make_async_copy(
          dst_hbm.at[pl.ds(0, CH)], dst_v[b], sem_i[b]).wait()
      pltpu.async_copy(g_hbm.at[src_v[b]], rows_v[b], sem_g[b])
    for b in range(SNB):
      pltpu.make_async_copy(
          g_hbm.at[src_v[b]], rows_v[b], sem_g[b]).wait()
      pltpu.async_copy(
          rows_v[b], acc_sh.at[dst_v[b]], sem_s[b], add=True)
    for b in range(SNB):
      pltpu.make_async_copy(
          rows_v[b], acc_sh.at[dst_v[b]], sem_s[b]).wait()
      nxt = j0 + SNB + b

      @pl.when(nxt < cnt)
      def _():
        eb = (base0 + nxt) * CH
        pltpu.async_copy(src_hbm.at[pl.ds(eb, CH)], src_v[b], sem_i[b])
        pltpu.async_copy(dst_hbm.at[pl.ds(eb, CH)], dst_v[b], sem_i[b])
    return 0

  lax.fori_loop(0, cnt // SNB, group_body, 0)
  plsc.subcore_barrier()
  pltpu.sync_copy(acc_sh.at[pl.ds(s * RPT, RPT)],
                  out_hbm.at[c, pl.ds(s * RPT, RPT)])


@functools.cache
def _sc_calls():
  mesh = plsc.VectorSubcoreMesh(
      core_axis_name="c", subcore_axis_name="s",
      num_cores=NC, num_subcores=NS)
  hist_call = pl.kernel(
      _hist_body,
      out_type=jax.ShapeDtypeStruct((NC, NPAD, D), jnp.float32),
      mesh=mesh,
      scratch_types=(
          [pltpu.VMEM((CH,), jnp.int32)] * HNB
          + [pltpu.VMEM((CH, D), jnp.float32)] * HNB
          + [pltpu.VMEM_SHARED((NPAD, D), jnp.float32)]
          + [pltpu.SemaphoreType.DMA] * (2 * HNB)
      ),
  )
  scatter_call = pl.kernel(
      _scatter_body,
      out_type=jax.ShapeDtypeStruct((NC, NPAD, D), jnp.float32),
      mesh=mesh,
      scratch_types=(
          [pltpu.VMEM((CH,), jnp.int32)] * (2 * SNB)
          + [pltpu.VMEM((CH, D), jnp.float32)] * SNB
          + [pltpu.VMEM_SHARED((NPAD, D), jnp.float32)]
          + [pltpu.SemaphoreType.DMA] * (3 * SNB)
      ),
  )
  return hist_call, scatter_call


BR = 2048
GRID = NPAD // BR


def _stage_b_body(x_ref, h0_ref, h1_ref, w1_ref, g1_ref, dinv_ref):
  deg = h0_ref[:, 0] + h1_ref[:, 0] + 1.0
  dinv = lax.rsqrt(deg)
  h = jnp.dot(x_ref[...], w1_ref[...], preferred_element_type=jnp.float32)
  g1_ref[...] = h * dinv[:, None]
  dinv_ref[...] = dinv


def _stage_b(x_p, h0, h1, W1):
  return pl.pallas_call(
      _stage_b_body,
      grid=(GRID,),
      in_specs=[
          pl.BlockSpec((BR, D), lambda i: (i, 0)),
          pl.BlockSpec((BR, D), lambda i: (i, 0)),
          pl.BlockSpec((BR, D), lambda i: (i, 0)),
          pl.BlockSpec((D, D), lambda i: (0, 0)),
      ],
      out_specs=[
          pl.BlockSpec((BR, D), lambda i: (i, 0)),
          pl.BlockSpec((BR,), lambda i: (i,)),
      ],
      out_shape=[
          jax.ShapeDtypeStruct((NPAD, D), jnp.float32),
          jax.ShapeDtypeStruct((NPAD,), jnp.float32),
      ],
  )(x_p, h0, h1, W1)


def _stage_d_body(p1a_ref, p1b_ref, g1_ref, dinv_ref, w2_ref, b1_ref, g2_ref):
  dinv = dinv_ref[...]
  conv = (p1a_ref[...] + p1b_ref[...] + g1_ref[...]) * dinv[:, None] \
      + b1_ref[...]
  h1 = jnp.where(conv >= 0, conv, 0.2 * conv)
  h = jnp.dot(h1, w2_ref[...], preferred_element_type=jnp.float32)
  g2_ref[...] = h * dinv[:, None]


def _stage_d(p1a, p1b, g1, dinv, W2, b1):
  return pl.pallas_call(
      _stage_d_body,
      grid=(GRID,),
      in_specs=[
          pl.BlockSpec((BR, D), lambda i: (i, 0)),
          pl.BlockSpec((BR, D), lambda i: (i, 0)),
          pl.BlockSpec((BR, D), lambda i: (i, 0)),
          pl.BlockSpec((BR,), lambda i: (i,)),
          pl.BlockSpec((D, D), lambda i: (0, 0)),
          pl.BlockSpec((1, D), lambda i: (0, 0)),
      ],
      out_specs=pl.BlockSpec((BR, D), lambda i: (i, 0)),
      out_shape=jax.ShapeDtypeStruct((NPAD, D), jnp.float32),
  )(p1a, p1b, g1, dinv, W2, b1)


def _stage_f_body(p2a_ref, p2b_ref, g2_ref, dinv_ref, b2_ref,
                  ctx_ref, wg_ref, bg_ref, wb_ref, out_ref):
  gate = jax.nn.sigmoid(
      jnp.dot(ctx_ref[...], wg_ref[...], preferred_element_type=jnp.float32)
      + bg_ref[...])
  bias = jnp.dot(ctx_ref[...], wb_ref[...],
                 preferred_element_type=jnp.float32)
  dinv = dinv_ref[...]
  conv = (p2a_ref[...] + p2b_ref[...] + g2_ref[...]) * dinv[:, None] \
      + b2_ref[...]
  out_ref[...] = conv * gate + bias


def _stage_f(p2a, p2b, g2, dinv, b2, ctx, Wg, bg, Wb):
  return pl.pallas_call(
      _stage_f_body,
      grid=(GRID,),
      in_specs=[
          pl.BlockSpec((BR, D), lambda i: (i, 0)),
          pl.BlockSpec((BR, D), lambda i: (i, 0)),
          pl.BlockSpec((BR, D), lambda i: (i, 0)),
          pl.BlockSpec((BR,), lambda i: (i,)),
          pl.BlockSpec((1, D), lambda i: (0, 0)),
          pl.BlockSpec((1, D), lambda i: (0, 0)),
          pl.BlockSpec((D, D), lambda i: (0, 0)),
          pl.BlockSpec((1, D), lambda i: (0, 0)),
          pl.BlockSpec((D, D), lambda i: (0, 0)),
      ],
      out_specs=pl.BlockSpec((BR, D), lambda i: (i, 0)),
      out_shape=jax.ShapeDtypeStruct((NPAD, D), jnp.float32),
  )(p2a, p2b, g2, dinv, b2, ctx, Wg, bg, Wb)


def kernel(x, edge_index, ctx, W1, b1, W2, b2, Wg, bg, Wb):
  hist_call, scatter_call = _sc_calls()
  src = edge_index[0]
  dst = edge_index[1]
  pad = EPAD - E
  src_p = jnp.concatenate([src, jnp.zeros((pad,), jnp.int32)])
  dst_p = jnp.concatenate([dst, jnp.full((pad,), N, jnp.int32)])
  x_p = jnp.pad(x, ((0, NPAD - N), (0, 0)))
  zeros_d = jnp.zeros((NPAD, D), jnp.float32)
  ones_d = jnp.ones((CH, D), jnp.float32)

  hist = hist_call(dst_p, ones_d, zeros_d)
  g1, dinv = _stage_b(x_p, hist[0], hist[1], W1)
  p1 = scatter_call(g1, src_p, dst_p, zeros_d)
  g2 = _stage_d(p1[0], p1[1], g1, dinv, W2, b1.reshape(1, D))
  p2 = scatter_call(g2, src_p, dst_p, zeros_d)
  out = _stage_f(p2[0], p2[1], g2, dinv, b2.reshape(1, D),
                 ctx, Wg, bg.reshape(1, D), Wb)
  return out[:N]


# speedup vs baseline: 1.1295x; 1.1295x over previous
import functools

import jax
import jax.numpy as jnp
from jax import lax
from jax.experimental import pallas as pl
from jax.experimental.pallas import tpu as pltpu
from jax.experimental.pallas import tpu_sc as plsc

N = 10000
E = 320000
D = 128

NC = 2
NS = 16
NW = NC * NS
L = 16

CH = 64
HNB = 2
SNB = 4
CPW = 160
EPAD = NW * CPW * CH
CW0 = 236
CW1 = 2 * CPW - CW0
NPAD = 10240
RPT = NPAD // NS


def _hist_body(dst_hbm, ones_hbm, zeros_hbm, out_hbm,
               dst_v0, dst_v1, ones_v0, ones_v1, acc_sh,
               si0, si1, ss0, ss1):
  dst_v = [dst_v0, dst_v1]
  ones_v = [ones_v0, ones_v1]
  sem_i = [si0, si1]
  sem_s = [ss0, ss1]
  c = lax.axis_index("c")
  s = lax.axis_index("s")
  wid = c * NS + s
  base0 = wid * CPW
  pltpu.sync_copy(zeros_hbm.at[pl.ds(s * RPT, RPT)],
                  acc_sh.at[pl.ds(s * RPT, RPT)])
  for b in range(HNB):
    pltpu.sync_copy(ones_hbm, ones_v[b])
  plsc.subcore_barrier()

  for b in range(HNB):
    eb = (base0 + b) * CH
    pltpu.async_copy(dst_hbm.at[pl.ds(eb, CH)], dst_v[b], sem_i[b])

  def group_body(i, _):
    j0 = i * HNB
    for b in range(HNB):
      pltpu.make_async_copy(
          dst_hbm.at[pl.ds(0, CH)], dst_v[b], sem_i[b]).wait()
      pltpu.async_copy(ones_v[b], acc_sh.at[dst_v[b]], sem_s[b], add=True)
    for b in range(HNB):
      pltpu.make_async_copy(
          ones_v[b], acc_sh.at[dst_v[b]], sem_s[b]).wait()
      nxt = j0 + HNB + b

      @pl.when(nxt < CPW)
      def _():
        eb = (base0 + nxt) * CH
        pltpu.async_copy(dst_hbm.at[pl.ds(eb, CH)], dst_v[b], sem_i[b])
    return 0

  lax.fori_loop(0, CPW // HNB, group_body, 0)
  plsc.subcore_barrier()
  pltpu.sync_copy(acc_sh.at[pl.ds(s * RPT, RPT)],
                  out_hbm.at[c, pl.ds(s * RPT, RPT)])


def _scatter_body(g_hbm, src_hbm, dst_hbm, zeros_hbm, out_hbm, *rest):
  src_v = list(rest[0:SNB])
  dst_v = list(rest[SNB:2 * SNB])
  rows_v = list(rest[2 * SNB:3 * SNB])
  acc_sh = rest[3 * SNB]
  sem_i = list(rest[3 * SNB + 1:3 * SNB + 1 + SNB])
  sem_g = list(rest[3 * SNB + 1 + SNB:3 * SNB + 1 + 2 * SNB])
  sem_s = list(rest[3 * SNB + 1 + 2 * SNB:3 * SNB + 1 + 3 * SNB])
  c = lax.axis_index("c")
  s = lax.axis_index("s")
  base0 = jnp.where(c == 0, s * CW0, NS * CW0 + s * CW1)
  cnt = jnp.where(c == 0, CW0, CW1)

  pltpu.sync_copy(zeros_hbm.at[pl.ds(s * RPT, RPT)],
                  acc_sh.at[pl.ds(s * RPT, RPT)])
  plsc.subcore_barrier()

  for b in range(SNB):
    eb = (base0 + b) * CH
    pltpu.async_copy(src_hbm.at[pl.ds(eb, CH)], src_v[b], sem_i[b])
    pltpu.async_copy(dst_hbm.at[pl.ds(eb, CH)], dst_v[b], sem_i[b])

  def group_body(i, _):
    j0 = i * SNB
    for b in range(SNB):
      pltpu.make_async_copy(
          src_hbm.at[pl.ds(0, CH)], src_v[b], sem_i[b]).wait()
      pltpu.make_async_copy(
          dst_hbm.at[pl.ds(0, CH)], dst_v[b], sem_i[b]).wait()
      pltpu.async_copy(g_hbm.at[src_v[b]], rows_v[b], sem_g[b])
    for b in range(SNB):
      pltpu.make_async_copy(
          g_hbm.at[src_v[b]], rows_v[b], sem_g[b]).wait()
      pltpu.async_copy(
          rows_v[b], acc_sh.at[dst_v[b]], sem_s[b], add=True)
    for b in range(SNB):
      pltpu.make_async_copy(
          rows_v[b], acc_sh.at[dst_v[b]], sem_s[b]).wait()
      nxt = j0 + SNB + b

      @pl.when(nxt < cnt)
      def _():
        eb = (base0 + nxt) * CH
        pltpu.async_copy(src_hbm.at[pl.ds(eb, CH)], src_v[b], sem_i[b])
        pltpu.async_copy(dst_hbm.at[pl.ds(eb, CH)], dst_v[b], sem_i[b])
    return 0

  lax.fori_loop(0, cnt // SNB, group_body, 0)
  plsc.subcore_barrier()
  pltpu.sync_copy(acc_sh.at[pl.ds(s * RPT, RPT)],
                  out_hbm.at[c, pl.ds(s * RPT, RPT)])


@functools.cache
def _sc_calls():
  mesh = plsc.VectorSubcoreMesh(
      core_axis_name="c", subcore_axis_name="s",
      num_cores=NC, num_subcores=NS)
  hist_call = pl.kernel(
      _hist_body,
      out_type=jax.ShapeDtypeStruct((NC, NPAD, D), jnp.float32),
      mesh=mesh,
      scratch_types=(
          [pltpu.VMEM((CH,), jnp.int32)] * HNB
          + [pltpu.VMEM((CH, D), jnp.float32)] * HNB
          + [pltpu.VMEM_SHARED((NPAD, D), jnp.float32)]
          + [pltpu.SemaphoreType.DMA] * (2 * HNB)
      ),
  )
  scatter_call = pl.kernel(
      _scatter_body,
      out_type=jax.ShapeDtypeStruct((NC, NPAD, D), jnp.float32),
      mesh=mesh,
      scratch_types=(
          [pltpu.VMEM((CH,), jnp.int32)] * (2 * SNB)
          + [pltpu.VMEM((CH, D), jnp.float32)] * SNB
          + [pltpu.VMEM_SHARED((NPAD, D), jnp.float32)]
          + [pltpu.SemaphoreType.DMA] * (3 * SNB)
      ),
  )
  return hist_call, scatter_call


BR = 2048
GRID = NPAD // BR


def _stage_b_body(x_ref, h0_ref, h1_ref, w1_ref, g1_ref, dinv_ref):
  deg = h0_ref[:, 0] + h1_ref[:, 0] + 1.0
  dinv = lax.rsqrt(deg)
  h = jnp.dot(x_ref[...], w1_ref[...], preferred_element_type=jnp.float32)
  g1_ref[...] = h * dinv[:, None]
  dinv_ref[...] = dinv


def _stage_b(x_p, h0, h1, W1):
  return pl.pallas_call(
      _stage_b_body,
      grid=(GRID,),
      in_specs=[
          pl.BlockSpec((BR, D), lambda i: (i, 0)),
          pl.BlockSpec((BR, D), lambda i: (i, 0)),
          pl.BlockSpec((BR, D), lambda i: (i, 0)),
          pl.BlockSpec((D, D), lambda i: (0, 0)),
      ],
      out_specs=[
          pl.BlockSpec((BR, D), lambda i: (i, 0)),
          pl.BlockSpec((BR,), lambda i: (i,)),
      ],
      out_shape=[
          jax.ShapeDtypeStruct((NPAD, D), jnp.float32),
          jax.ShapeDtypeStruct((NPAD,), jnp.float32),
      ],
  )(x_p, h0, h1, W1)


def _stage_d_body(p1a_ref, p1b_ref, g1_ref, dinv_ref, w2_ref, b1_ref, g2_ref):
  dinv = dinv_ref[...]
  conv = (p1a_ref[...] + p1b_ref[...] + g1_ref[...]) * dinv[:, None] \
      + b1_ref[...]
  h1 = jnp.where(conv >= 0, conv, 0.2 * conv)
  h = jnp.dot(h1, w2_ref[...], preferred_element_type=jnp.float32)
  g2_ref[...] = h * dinv[:, None]


def _stage_d(p1a, p1b, g1, dinv, W2, b1):
  return pl.pallas_call(
      _stage_d_body,
      grid=(GRID,),
      in_specs=[
          pl.BlockSpec((BR, D), lambda i: (i, 0)),
          pl.BlockSpec((BR, D), lambda i: (i, 0)),
          pl.BlockSpec((BR, D), lambda i: (i, 0)),
          pl.BlockSpec((BR,), lambda i: (i,)),
          pl.BlockSpec((D, D), lambda i: (0, 0)),
          pl.BlockSpec((1, D), lambda i: (0, 0)),
      ],
      out_specs=pl.BlockSpec((BR, D), lambda i: (i, 0)),
      out_shape=jax.ShapeDtypeStruct((NPAD, D), jnp.float32),
  )(p1a, p1b, g1, dinv, W2, b1)


def _stage_f_body(p2a_ref, p2b_ref, g2_ref, dinv_ref, b2_ref,
                  ctx_ref, wg_ref, bg_ref, wb_ref, out_ref):
  gate = jax.nn.sigmoid(
      jnp.dot(ctx_ref[...], wg_ref[...], preferred_element_type=jnp.float32)
      + bg_ref[...])
  bias = jnp.dot(ctx_ref[...], wb_ref[...],
                 preferred_element_type=jnp.float32)
  dinv = dinv_ref[...]
  conv = (p2a_ref[...] + p2b_ref[...] + g2_ref[...]) * dinv[:, None] \
      + b2_ref[...]
  out_ref[...] = conv * gate + bias


def _stage_f(p2a, p2b, g2, dinv, b2, ctx, Wg, bg, Wb):
  return pl.pallas_call(
      _stage_f_body,
      grid=(GRID,),
      in_specs=[
          pl.BlockSpec((BR, D), lambda i: (i, 0)),
          pl.BlockSpec((BR, D), lambda i: (i, 0)),
          pl.BlockSpec((BR, D), lambda i: (i, 0)),
          pl.BlockSpec((BR,), lambda i: (i,)),
          pl.BlockSpec((1, D), lambda i: (0, 0)),
          pl.BlockSpec((1, D), lambda i: (0, 0)),
          pl.BlockSpec((D, D), lambda i: (0, 0)),
          pl.BlockSpec((1, D), lambda i: (0, 0)),
          pl.BlockSpec((D, D), lambda i: (0, 0)),
      ],
      out_specs=pl.BlockSpec((BR, D), lambda i: (i, 0)),
      out_shape=jax.ShapeDtypeStruct((NPAD, D), jnp.float32),
  )(p2a, p2b, g2, dinv, b2, ctx, Wg, bg, Wb)


def kernel(x, edge_index, ctx, W1, b1, W2, b2, Wg, bg, Wb):
  hist_call, scatter_call = _sc_calls()
  src = edge_index[0]
  dst = edge_index[1]
  pad = EPAD - E
  src_p = jnp.concatenate([src, jnp.zeros((pad,), jnp.int32)])
  dst_p = jnp.concatenate([dst, jnp.full((pad,), N, jnp.int32)])
  x_p = jnp.pad(x, ((0, NPAD - N), (0, 0)))
  zeros_d = jnp.zeros((NPAD, D), jnp.float32)
  ones_d = jnp.ones((CH, D), jnp.float32)

  hist = hist_call(dst_p, ones_d, zeros_d)
  g1, dinv = _stage_b(x_p, hist[0], hist[1], W1)
  p1 = scatter_call(g1, src_p, dst_p, zeros_d)
  g2 = _stage_d(p1[0], p1[1], g1, dinv, W2, b1.reshape(1, D))
  p2 = scatter_call(g2, src_p, dst_p, zeros_d)
  out = _stage_f(p2[0], p2[1], g2, dinv, b2.reshape(1, D),
                 ctx, Wg, bg.reshape(1, D), Wb)
  return out[:N]


# speedup vs baseline: 2.3716x; 2.0997x over previous
import functools

import jax
import jax.numpy as jnp
from jax import lax
from jax.experimental import pallas as pl
from jax.experimental.pallas import tpu as pltpu
from jax.experimental.pallas import tpu_sc as plsc

N = 10000
E = 320000
D = 128

NC = 2
NS = 16
NW = NC * NS
L = 16

CH = 64
HNB = 2
SNB = 4
CPW = 160
EPAD = NW * CPW * CH
CW0 = CPW
CW1 = 2 * CPW - CW0
NPAD = 10240
RPT = NPAD // NS


def _hist_body(dst_hbm, ones_hbm, zeros_hbm, out_hbm,
               dst_v0, dst_v1, ones_v0, ones_v1, acc_sh,
               si0, si1, ss0, ss1):
  dst_v = [dst_v0, dst_v1]
  ones_v = [ones_v0, ones_v1]
  sem_i = [si0, si1]
  sem_s = [ss0, ss1]
  c = lax.axis_index("c")
  s = lax.axis_index("s")
  wid = c * NS + s
  base0 = wid * CPW
  pltpu.sync_copy(zeros_hbm.at[pl.ds(s * RPT, RPT)],
                  acc_sh.at[pl.ds(s * RPT, RPT)])
  for b in range(HNB):
    pltpu.sync_copy(ones_hbm, ones_v[b])
  plsc.subcore_barrier()

  for b in range(HNB):
    eb = (base0 + b) * CH
    pltpu.async_copy(dst_hbm.at[pl.ds(eb, CH)], dst_v[b], sem_i[b])

  def group_body(i, _):
    j0 = i * HNB
    for b in range(HNB):
      pltpu.make_async_copy(
          dst_hbm.at[pl.ds(0, CH)], dst_v[b], sem_i[b]).wait()
      pltpu.async_copy(ones_v[b], acc_sh.at[dst_v[b]], sem_s[b], add=True)
    for b in range(HNB):
      pltpu.make_async_copy(
          ones_v[b], acc_sh.at[dst_v[b]], sem_s[b]).wait()
      nxt = j0 + HNB + b

      @pl.when(nxt < CPW)
      def _():
        eb = (base0 + nxt) * CH
        pltpu.async_copy(dst_hbm.at[pl.ds(eb, CH)], dst_v[b], sem_i[b])
    return 0

  lax.fori_loop(0, CPW // HNB, group_body, 0)
  plsc.subcore_barrier()
  pltpu.sync_copy(acc_sh.at[pl.ds(s * RPT, RPT)],
                  out_hbm.at[c, pl.ds(s * RPT, RPT)])


def _scatter_body(g_hbm, src_hbm, dst_hbm, zeros_hbm, out_hbm, *rest):
  src_v = list(rest[0:SNB])
  dst_v = list(rest[SNB:2 * SNB])
  rows_v = list(rest[2 * SNB:3 * SNB])
  acc_sh = rest[3 * SNB]
  sem_i = list(rest[3 * SNB + 1:3 * SNB + 1 + SNB])
  sem_g = list(rest[3 * SNB + 1 + SNB:3 * SNB + 1 + 2 * SNB])
  sem_s = list(rest[3 * SNB + 1 + 2 * SNB:3 * SNB + 1 + 3 * SNB])
  c = lax.axis_index("c")
  s = lax.axis_index("s")
  base0 = jnp.where(c == 0, s * CW0, NS * CW0 + s * CW1)
  cnt = jnp.where(c == 0, CW0, CW1)

  pltpu.sync_copy(zeros_hbm.at[pl.ds(s * RPT, RPT)],
                  acc_sh.at[pl.ds(s * RPT, RPT)])
  plsc.subcore_barrier()

  for b in range(SNB):
    eb = (base0 + b) * CH
    pltpu.async_copy(src_hbm.at[pl.ds(eb, CH)], src_v[b], sem_i[b])
    pltpu.async_copy(dst_hbm.at[pl.ds(eb, CH)], dst_v[b], sem_i[b])

  def group_body(i, _):
    j0 = i * SNB
    for b in range(SNB):
      pltpu.make_async_copy(
          src_hbm.at[pl.ds(0, CH)], src_v[b], sem_i[b]).wait()
      pltpu.make_async_copy(
          dst_hbm.at[pl.ds(0, CH)], dst_v[b], sem_i[b]).wait()
      pltpu.async_copy(g_hbm.at[src_v[b]], rows_v[b], sem_g[b])
    for b in range(SNB):
      pltpu.make_async_copy(
          g_hbm.at[src_v[b]], rows_v[b], sem_g[b]).wait()
      pltpu.async_copy(
          rows_v[b], acc_sh.at[dst_v[b]], sem_s[b], add=True)
    for b in range(SNB):
      pltpu.make_async_copy(
          rows_v[b], acc_sh.at[dst_v[b]], sem_s[b]).wait()
      nxt = j0 + SNB + b

      @pl.when(nxt < cnt)
      def _():
        eb = (base0 + nxt) * CH
        pltpu.async_copy(src_hbm.at[pl.ds(eb, CH)], src_v[b], sem_i[b])
        pltpu.async_copy(dst_hbm.at[pl.ds(eb, CH)], dst_v[b], sem_i[b])
    return 0

  lax.fori_loop(0, cnt // SNB, group_body, 0)
  plsc.subcore_barrier()
  pltpu.sync_copy(acc_sh.at[pl.ds(s * RPT, RPT)],
                  out_hbm.at[c, pl.ds(s * RPT, RPT)])


@functools.cache
def _sc_calls():
  mesh = plsc.VectorSubcoreMesh(
      core_axis_name="c", subcore_axis_name="s",
      num_cores=NC, num_subcores=NS)
  hist_call = pl.kernel(
      _hist_body,
      out_type=jax.ShapeDtypeStruct((NC, NPAD, D), jnp.float32),
      mesh=mesh,
      scratch_types=(
          [pltpu.VMEM((CH,), jnp.int32)] * HNB
          + [pltpu.VMEM((CH, D), jnp.float32)] * HNB
          + [pltpu.VMEM_SHARED((NPAD, D), jnp.float32)]
          + [pltpu.SemaphoreType.DMA] * (2 * HNB)
      ),
  )
  scatter_call = pl.kernel(
      _scatter_body,
      out_type=jax.ShapeDtypeStruct((NC, NPAD, D), jnp.float32),
      mesh=mesh,
      scratch_types=(
          [pltpu.VMEM((CH,), jnp.int32)] * (2 * SNB)
          + [pltpu.VMEM((CH, D), jnp.float32)] * SNB
          + [pltpu.VMEM_SHARED((NPAD, D), jnp.float32)]
          + [pltpu.SemaphoreType.DMA] * (3 * SNB)
      ),
  )
  return hist_call, scatter_call


BR = 2048
GRID = NPAD // BR


def _stage_b_body(x_ref, h0_ref, h1_ref, w1_ref, g1_ref, dinv_ref):
  deg = h0_ref[:, 0] + h1_ref[:, 0] + 1.0
  dinv = lax.rsqrt(deg)
  h = jnp.dot(x_ref[...], w1_ref[...], preferred_element_type=jnp.float32)
  g1_ref[...] = h * dinv[:, None]
  dinv_ref[...] = dinv


def _stage_b(x_p, h0, h1, W1):
  return pl.pallas_call(
      _stage_b_body,
      grid=(GRID,),
      in_specs=[
          pl.BlockSpec((BR, D), lambda i: (i, 0)),
          pl.BlockSpec((BR, D), lambda i: (i, 0)),
          pl.BlockSpec((BR, D), lambda i: (i, 0)),
          pl.BlockSpec((D, D), lambda i: (0, 0)),
      ],
      out_specs=[
          pl.BlockSpec((BR, D), lambda i: (i, 0)),
          pl.BlockSpec((BR,), lambda i: (i,)),
      ],
      out_shape=[
          jax.ShapeDtypeStruct((NPAD, D), jnp.float32),
          jax.ShapeDtypeStruct((NPAD,), jnp.float32),
      ],
  )(x_p, h0, h1, W1)


def _stage_d_body(p1a_ref, p1b_ref, g1_ref, dinv_ref, w2_ref, b1_ref, g2_ref):
  dinv = dinv_ref[...]
  conv = (p1a_ref[...] + p1b_ref[...] + g1_ref[...]) * dinv[:, None] \
      + b1_ref[...]
  h1 = jnp.where(conv >= 0, conv, 0.2 * conv)
  h = jnp.dot(h1, w2_ref[...], preferred_element_type=jnp.float32)
  g2_ref[...] = h * dinv[:, None]


def _stage_d(p1a, p1b, g1, dinv, W2, b1):
  return pl.pallas_call(
      _stage_d_body,
      grid=(GRID,),
      in_specs=[
          pl.BlockSpec((BR, D), lambda i: (i, 0)),
          pl.BlockSpec((BR, D), lambda i: (i, 0)),
          pl.BlockSpec((BR, D), lambda i: (i, 0)),
          pl.BlockSpec((BR,), lambda i: (i,)),
          pl.BlockSpec((D, D), lambda i: (0, 0)),
          pl.BlockSpec((1, D), lambda i: (0, 0)),
      ],
      out_specs=pl.BlockSpec((BR, D), lambda i: (i, 0)),
      out_shape=jax.ShapeDtypeStruct((NPAD, D), jnp.float32),
  )(p1a, p1b, g1, dinv, W2, b1)


def _stage_f_body(p2a_ref, p2b_ref, g2_ref, dinv_ref, b2_ref,
                  ctx_ref, wg_ref, bg_ref, wb_ref, out_ref):
  gate = jax.nn.sigmoid(
      jnp.dot(ctx_ref[...], wg_ref[...], preferred_element_type=jnp.float32)
      + bg_ref[...])
  bias = jnp.dot(ctx_ref[...], wb_ref[...],
                 preferred_element_type=jnp.float32)
  dinv = dinv_ref[...]
  conv = (p2a_ref[...] + p2b_ref[...] + g2_ref[...]) * dinv[:, None] \
      + b2_ref[...]
  out_ref[...] = conv * gate + bias


def _stage_f(p2a, p2b, g2, dinv, b2, ctx, Wg, bg, Wb):
  return pl.pallas_call(
      _stage_f_body,
      grid=(GRID,),
      in_specs=[
          pl.BlockSpec((BR, D), lambda i: (i, 0)),
          pl.BlockSpec((BR, D), lambda i: (i, 0)),
          pl.BlockSpec((BR, D), lambda i: (i, 0)),
          pl.BlockSpec((BR,), lambda i: (i,)),
          pl.BlockSpec((1, D), lambda i: (0, 0)),
          pl.BlockSpec((1, D), lambda i: (0, 0)),
          pl.BlockSpec((D, D), lambda i: (0, 0)),
          pl.BlockSpec((1, D), lambda i: (0, 0)),
          pl.BlockSpec((D, D), lambda i: (0, 0)),
      ],
      out_specs=pl.BlockSpec((BR, D), lambda i: (i, 0)),
      out_shape=jax.ShapeDtypeStruct((NPAD, D), jnp.float32),
  )(p2a, p2b, g2, dinv, b2, ctx, Wg, bg, Wb)


def kernel(x, edge_index, ctx, W1, b1, W2, b2, Wg, bg, Wb):
  hist_call, scatter_call = _sc_calls()
  src = edge_index[0]
  dst = edge_index[1]
  pad = EPAD - E
  padidx = N + jnp.arange(pad, dtype=jnp.int32) % (NPAD - N - 1)
  src_p = jnp.concatenate([src, padidx])
  dst_p = jnp.concatenate([dst, padidx])
  x_p = jnp.pad(x, ((0, NPAD - N), (0, 0)))
  zeros_d = jnp.zeros((NPAD, D), jnp.float32)
  ones_d = jnp.ones((CH, D), jnp.float32)

  hist = hist_call(dst_p, ones_d, zeros_d)
  g1, dinv = _stage_b(x_p, hist[0], hist[1], W1)
  p1 = scatter_call(g1, src_p, dst_p, zeros_d)
  g2 = _stage_d(p1[0], p1[1], g1, dinv, W2, b1.reshape(1, D))
  p2 = scatter_call(g2, src_p, dst_p, zeros_d)
  out = _stage_f(p2[0], p2[1], g2, dinv, b2.reshape(1, D),
                 ctx, Wg, bg.reshape(1, D), Wb)
  return out[:N]
